# baseline (speedup 1.0000x reference)
import functools

import jax
import jax.numpy as jnp
from jax import lax
from jax.experimental import pallas as pl
from jax.experimental.pallas import tpu as pltpu
from jax.experimental.pallas import tpu_sc as plsc

N = 10000
D = 128
E = 320000
NC = 2
NS = 16
B = 96
NG = 15
NGR = 7
CH = NG * NGR
T_TILE = B * CH
E_PAD = NC * NS * T_TILE
R = 10112
RPT = R // NS
RD = 10240
RDPT = RD // NS
BLK = 2528

_mesh = plsc.VectorSubcoreMesh(
    core_axis_name="c", subcore_axis_name="s", num_cores=NC, num_subcores=NS
)


@functools.partial(
    pl.kernel,
    out_type=jax.ShapeDtypeStruct((NC, RD), jnp.float32),
    mesh=_mesh,
    scratch_types=[
        pltpu.VMEM((NGR, NG, B), jnp.int32),
        pltpu.VMEM((B,), jnp.float32),
        pltpu.VMEM_SHARED((RD,), jnp.float32),
        pltpu.SemaphoreType.DMA,
    ],
)
def _deg_kernel(dst_hbm, zeros1_hbm, out_hbm, dstv, ones_v, dacc, dsem):
    c = lax.axis_index("c")
    s = lax.axis_index("s")
    r0 = s * RDPT
    pltpu.sync_copy(zeros1_hbm.at[pl.ds(r0, RDPT)], dacc.at[pl.ds(r0, RDPT)])
    pltpu.sync_copy(dst_hbm.at[c, s], dstv)
    for k in range(B // 16):
        ones_v[pl.ds(16 * k, 16)] = jnp.ones((16,), jnp.float32)
    plsc.subcore_barrier()

    for g in range(NGR):

        def body(j, _):
            pltpu.async_copy(ones_v, dacc.at[dstv.at[g, j]], dsem, add=True)
            return ()

        lax.fori_loop(0, NG, body, ())

    def drain(j, _):
        pltpu.make_async_copy(ones_v, dacc.at[dstv.at[0, 0]], dsem).wait()
        return ()

    lax.fori_loop(0, NGR * NG, drain, ())
    plsc.subcore_barrier()
    pltpu.sync_copy(dacc.at[pl.ds(r0, RDPT)], out_hbm.at[c, pl.ds(r0, RDPT)])


@functools.partial(
    pl.kernel,
    out_type=jax.ShapeDtypeStruct((NC, R, D), jnp.float32),
    mesh=_mesh,
    scratch_types=[
        pltpu.VMEM((2, NG, B), jnp.int32),
        pltpu.VMEM((2, NG, B), jnp.int32),
        pltpu.VMEM((B, D), jnp.float32),
        pltpu.VMEM((B, D), jnp.float32),
        pltpu.VMEM((B, D), jnp.float32),
        pltpu.VMEM_SHARED((R, D), jnp.float32),
        pltpu.SemaphoreType.DMA,
        pltpu.SemaphoreType.DMA,
        pltpu.SemaphoreType.DMA,
        pltpu.SemaphoreType.DMA,
        pltpu.SemaphoreType.DMA,
    ],
)
def _spmm_kernel(t_hbm, src_hbm, dst_hbm, zeros2_hbm, out_hbm,
                 gsrc, gdst, rows0, rows1, rows2, acc,
                 sem0, sem1, sem2, si0, si1):
    c = lax.axis_index("c")
    s = lax.axis_index("s")
    r0 = s * RPT

    @pl.when(c == 0)
    def _():
        pltpu.sync_copy(t_hbm.at[pl.ds(r0, RPT)], acc.at[pl.ds(r0, RPT)])

    @pl.when(c != 0)
    def _():
        pltpu.sync_copy(zeros2_hbm.at[pl.ds(r0, RPT)], acc.at[pl.ds(r0, RPT)])

    sis = (si0, si1)

    def idx_load(g):
        b = g % 2
        pltpu.async_copy(src_hbm.at[c, s, g], gsrc.at[b], sis[b])
        pltpu.async_copy(dst_hbm.at[c, s, g], gdst.at[b], sis[b])

    def idx_wait(g):
        b = g % 2
        pltpu.make_async_copy(src_hbm.at[c, s, g], gsrc.at[b], sis[b]).wait()
        pltpu.make_async_copy(dst_hbm.at[c, s, g], gdst.at[b], sis[b]).wait()

    def gstart(sg, j, rows, sem):
        pltpu.async_copy(t_hbm.at[sg.at[j]], rows, sem)

    def gwait(sg, j, rows, sem):
        pltpu.make_async_copy(t_hbm.at[sg.at[j]], rows, sem).wait()

    def scat(dg, j, rows):
        pltpu.sync_copy(rows, acc.at[dg.at[j]], add=True)

    idx_load(0)
    idx_wait(0)
    idx_load(1)
    plsc.subcore_barrier()

    bufs = ((rows0, sem0), (rows1, sem1), (rows2, sem2))
    sg0 = gsrc.at[0]
    gstart(sg0, 0, rows0, sem0)
    gstart(sg0, 1, rows1, sem1)

    for g in range(NGR):
        bb = g % 2
        sg, dg = gsrc.at[bb], gdst.at[bb]

        def triple(i, _):
            j0 = 3 * i
            for k in range(3):
                rows, sem = bufs[k]
                nrows, nsem = bufs[(k + 2) % 3]
                gwait(sg, j0 + k, rows, sem)
                gstart(sg, j0 + k + 2, nrows, nsem)
                scat(dg, j0 + k, rows)
            return ()

        lax.fori_loop(0, NG // 3 - 1, triple, ())
        for k in range(3):
            j = NG - 3 + k
            rows, sem = bufs[j % 3]
            nrows, nsem = bufs[(j + 2) % 3]
            gwait(sg, j, rows, sem)
            if k == 0:
                gstart(sg, NG - 1, nrows, nsem)
            elif g + 1 < NGR:
                if k == 1:
                    idx_wait(g + 1)
                gstart(gsrc.at[(g + 1) % 2], k - 1, nrows, nsem)
            scat(dg, j, rows)
        if g + 2 < NGR:
            idx_load(g + 2)

    plsc.subcore_barrier()
    pltpu.sync_copy(acc.at[pl.ds(r0, RPT)], out_hbm.at[c, pl.ds(r0, RPT)])


def _tc1_body(x_ref, w_ref, dg_ref, o_ref):
    dinv = lax.rsqrt(dg_ref[...])
    h = jnp.dot(x_ref[...], w_ref[...], preferred_element_type=jnp.float32)
    o_ref[...] = h * dinv


def _tc2_body(s_ref, dg_ref, b_ref, o_ref):
    pid = pl.program_id(0)
    rows = pid * BLK + lax.broadcasted_iota(jnp.int32, (BLK, 1), 0)
    dinv = lax.rsqrt(dg_ref[...])
    pre = dinv * (s_ref[0] + s_ref[1]) + b_ref[...]
    t2 = dinv * jnp.maximum(pre, 0.0)
    o_ref[...] = jnp.where(rows < N, t2, 0.0)


def _tc3_body(s_ref, dg_ref, w_ref, b_ref, o_ref):
    dinv = lax.rsqrt(dg_ref[...])
    agg = dinv * (s_ref[0] + s_ref[1])
    o_ref[...] = (
        jnp.dot(agg, w_ref[...], preferred_element_type=jnp.float32)
        + b_ref[...]
    )


_row_spec = pl.BlockSpec((BLK, D), lambda i: (i, 0))
_deg_spec = pl.BlockSpec((BLK, 1), lambda i: (i, 0))
_par_spec = pl.BlockSpec((NC, BLK, D), lambda i: (0, i, 0))
_w_spec = pl.BlockSpec((D, D), lambda i: (0, 0))
_b_spec = pl.BlockSpec((1, D), lambda i: (0, 0))
_out_row = jax.ShapeDtypeStruct((R, D), jnp.float32)

_tc1 = pl.pallas_call(
    _tc1_body, grid=(R // BLK,),
    in_specs=[_row_spec, _w_spec, _deg_spec],
    out_specs=_row_spec, out_shape=_out_row,
)
_tc2 = pl.pallas_call(
    _tc2_body, grid=(R // BLK,),
    in_specs=[_par_spec, _deg_spec, _b_spec],
    out_specs=_row_spec, out_shape=_out_row,
)
_tc3 = pl.pallas_call(
    _tc3_body, grid=(R // BLK,),
    in_specs=[_par_spec, _deg_spec, _w_spec, _b_spec],
    out_specs=_row_spec, out_shape=_out_row,
)


@jax.jit
def _pipeline(x, edge_index, W1, b1, Wmu, bmu, Wlv, blv):
    src = edge_index[0]
    dst = edge_index[1]
    pad = E_PAD - E
    pad_idx = (N + (jnp.arange(pad, dtype=jnp.int32) % (R - N))).astype(jnp.int32)
    src_p = jnp.concatenate([src, pad_idx]).reshape(NC, NS, NGR, NG, B)
    dst_p = jnp.concatenate([dst, pad_idx]).reshape(NC, NS, NGR, NG, B)

    x_p = jnp.zeros((R, D), x.dtype).at[:N].set(x)
    zeros1 = jnp.zeros((RD,), jnp.float32)
    zeros2 = jnp.zeros((R, D), jnp.float32)

    degp = _deg_kernel(dst_p, zeros1)
    dg = (degp[0, :R] + degp[1, :R] + 1.0).reshape(R, 1)

    t1 = _tc1(x_p, W1, dg)
    s1 = _spmm_kernel(t1, src_p, dst_p, zeros2)
    t2 = _tc2(s1, dg, b1.reshape(1, D))
    s2 = _spmm_kernel(t2, src_p, dst_p, zeros2)
    wcat = jnp.concatenate([Wmu, Wlv], axis=1)
    bcat = jnp.concatenate([bmu, blv]).reshape(1, D)
    out = _tc3(s2, dg, wcat, bcat)
    return out[:N, : D // 2], out[:N, D // 2 :]


def kernel(x, edge_index, W1, b1, Wmu, bmu, Wlv, blv):
    return _pipeline(x, edge_index, W1, b1, Wmu, bmu, Wlv, blv)


# speedup vs baseline: 1.0237x; 1.0237x over previous
import functools

import jax
import jax.numpy as jnp
from jax import lax
from jax.experimental import pallas as pl
from jax.experimental.pallas import tpu as pltpu
from jax.experimental.pallas import tpu_sc as plsc

N = 10000
D = 128
E = 320000
NC = 2
NS = 16
B = 64
NB = 4
NG = 20
NGR = 8
CH = NG * NGR
T_TILE = B * CH
E_PAD = NC * NS * T_TILE
R = 10112
RPT = R // NS
RD = 10240
RDPT = RD // NS
BLK = 2528

_mesh = plsc.VectorSubcoreMesh(
    core_axis_name="c", subcore_axis_name="s", num_cores=NC, num_subcores=NS
)


@functools.partial(
    pl.kernel,
    out_type=jax.ShapeDtypeStruct((NC, RD), jnp.float32),
    mesh=_mesh,
    scratch_types=[
        pltpu.VMEM((NGR, NG, B), jnp.int32),
        pltpu.VMEM((B,), jnp.float32),
        pltpu.VMEM_SHARED((RD,), jnp.float32),
        pltpu.SemaphoreType.DMA,
    ],
)
def _deg_kernel(dst_hbm, zeros1_hbm, out_hbm, dstv, ones_v, dacc, dsem):
    c = lax.axis_index("c")
    s = lax.axis_index("s")
    r0 = s * RDPT
    pltpu.sync_copy(zeros1_hbm.at[pl.ds(r0, RDPT)], dacc.at[pl.ds(r0, RDPT)])
    pltpu.sync_copy(dst_hbm.at[c, s], dstv)
    for k in range(B // 16):
        ones_v[pl.ds(16 * k, 16)] = jnp.ones((16,), jnp.float32)
    plsc.subcore_barrier()

    for g in range(NGR):

        def body(j, _):
            pltpu.async_copy(ones_v, dacc.at[dstv.at[g, j]], dsem, add=True)
            return ()

        lax.fori_loop(0, NG, body, ())

    def drain(j, _):
        pltpu.make_async_copy(ones_v, dacc.at[dstv.at[0, 0]], dsem).wait()
        return ()

    lax.fori_loop(0, NGR * NG, drain, ())
    plsc.subcore_barrier()
    pltpu.sync_copy(dacc.at[pl.ds(r0, RDPT)], out_hbm.at[c, pl.ds(r0, RDPT)])


@functools.partial(
    pl.kernel,
    out_type=jax.ShapeDtypeStruct((NC, R, D), jnp.float32),
    mesh=_mesh,
    scratch_types=[
        pltpu.VMEM((2, NG, B), jnp.int32),
        pltpu.VMEM((2, NG, B), jnp.int32),
        pltpu.VMEM((B, D), jnp.float32),
        pltpu.VMEM((B, D), jnp.float32),
        pltpu.VMEM((B, D), jnp.float32),
        pltpu.VMEM((B, D), jnp.float32),
        pltpu.VMEM_SHARED((R, D), jnp.float32),
        pltpu.SemaphoreType.DMA,
        pltpu.SemaphoreType.DMA,
        pltpu.SemaphoreType.DMA,
        pltpu.SemaphoreType.DMA,
        pltpu.SemaphoreType.DMA,
        pltpu.SemaphoreType.DMA,
    ],
)
def _spmm_kernel(t_hbm, src_hbm, dst_hbm, zeros2_hbm, out_hbm,
                 gsrc, gdst, rows0, rows1, rows2, rows3, acc,
                 sem0, sem1, sem2, sem3, si0, si1):
    c = lax.axis_index("c")
    s = lax.axis_index("s")
    r0 = s * RPT

    @pl.when(c == 0)
    def _():
        pltpu.sync_copy(t_hbm.at[pl.ds(r0, RPT)], acc.at[pl.ds(r0, RPT)])

    @pl.when(c != 0)
    def _():
        pltpu.sync_copy(zeros2_hbm.at[pl.ds(r0, RPT)], acc.at[pl.ds(r0, RPT)])

    sis = (si0, si1)

    def idx_load(g):
        b = g % 2
        pltpu.async_copy(src_hbm.at[c, s, g], gsrc.at[b], sis[b])
        pltpu.async_copy(dst_hbm.at[c, s, g], gdst.at[b], sis[b])

    def idx_wait(g):
        b = g % 2
        pltpu.make_async_copy(src_hbm.at[c, s, g], gsrc.at[b], sis[b]).wait()
        pltpu.make_async_copy(dst_hbm.at[c, s, g], gdst.at[b], sis[b]).wait()

    def gstart(sg, j, rows, sem):
        pltpu.async_copy(t_hbm.at[sg.at[j]], rows, sem)

    def gwait(sg, j, rows, sem):
        pltpu.make_async_copy(t_hbm.at[sg.at[j]], rows, sem).wait()

    def scat(dg, j, rows):
        pltpu.sync_copy(rows, acc.at[dg.at[j]], add=True)

    idx_load(0)
    idx_wait(0)
    idx_load(1)
    plsc.subcore_barrier()

    bufs = ((rows0, sem0), (rows1, sem1), (rows2, sem2), (rows3, sem3))
    sg0 = gsrc.at[0]
    gstart(sg0, 0, rows0, sem0)
    gstart(sg0, 1, rows1, sem1)
    gstart(sg0, 2, rows2, sem2)

    for g in range(NGR):
        bb = g % 2
        sg, dg = gsrc.at[bb], gdst.at[bb]

        def quad(i, _):
            j0 = NB * i
            for k in range(NB):
                rows, sem = bufs[k]
                nrows, nsem = bufs[(k + 3) % NB]
                gwait(sg, j0 + k, rows, sem)
                gstart(sg, j0 + k + 3, nrows, nsem)
                scat(dg, j0 + k, rows)
            return ()

        lax.fori_loop(0, NG // NB - 1, quad, ())
        for k in range(NB):
            j = NG - NB + k
            rows, sem = bufs[j % NB]
            nrows, nsem = bufs[(j + 3) % NB]
            gwait(sg, j, rows, sem)
            if k == 0:
                gstart(sg, NG - 1, nrows, nsem)
            elif g + 1 < NGR:
                if k == 1:
                    idx_wait(g + 1)
                gstart(gsrc.at[(g + 1) % 2], k - 1, nrows, nsem)
            scat(dg, j, rows)
        if g + 2 < NGR:
            idx_load(g + 2)

    plsc.subcore_barrier()
    pltpu.sync_copy(acc.at[pl.ds(r0, RPT)], out_hbm.at[c, pl.ds(r0, RPT)])


def _tc1_body(x_ref, w_ref, dg_ref, o_ref):
    dinv = lax.rsqrt(dg_ref[...])
    h = jnp.dot(x_ref[...], w_ref[...], preferred_element_type=jnp.float32)
    o_ref[...] = h * dinv


def _tc2_body(s_ref, dg_ref, b_ref, o_ref):
    pid = pl.program_id(0)
    rows = pid * BLK + lax.broadcasted_iota(jnp.int32, (BLK, 1), 0)
    dinv = lax.rsqrt(dg_ref[...])
    pre = dinv * (s_ref[0] + s_ref[1]) + b_ref[...]
    t2 = dinv * jnp.maximum(pre, 0.0)
    o_ref[...] = jnp.where(rows < N, t2, 0.0)


def _tc3_body(s_ref, dg_ref, w_ref, b_ref, o_ref):
    dinv = lax.rsqrt(dg_ref[...])
    agg = dinv * (s_ref[0] + s_ref[1])
    o_ref[...] = (
        jnp.dot(agg, w_ref[...], preferred_element_type=jnp.float32)
        + b_ref[...]
    )


_row_spec = pl.BlockSpec((BLK, D), lambda i: (i, 0))
_deg_spec = pl.BlockSpec((BLK, 1), lambda i: (i, 0))
_par_spec = pl.BlockSpec((NC, BLK, D), lambda i: (0, i, 0))
_w_spec = pl.BlockSpec((D, D), lambda i: (0, 0))
_b_spec = pl.BlockSpec((1, D), lambda i: (0, 0))
_out_row = jax.ShapeDtypeStruct((R, D), jnp.float32)

_tc1 = pl.pallas_call(
    _tc1_body, grid=(R // BLK,),
    in_specs=[_row_spec, _w_spec, _deg_spec],
    out_specs=_row_spec, out_shape=_out_row,
)
_tc2 = pl.pallas_call(
    _tc2_body, grid=(R // BLK,),
    in_specs=[_par_spec, _deg_spec, _b_spec],
    out_specs=_row_spec, out_shape=_out_row,
)
_tc3 = pl.pallas_call(
    _tc3_body, grid=(R // BLK,),
    in_specs=[_par_spec, _deg_spec, _w_spec, _b_spec],
    out_specs=_row_spec, out_shape=_out_row,
)


@jax.jit
def _pipeline(x, edge_index, W1, b1, Wmu, bmu, Wlv, blv):
    src = edge_index[0]
    dst = edge_index[1]
    pad = E_PAD - E
    pad_idx = (N + (jnp.arange(pad, dtype=jnp.int32) % (R - N))).astype(jnp.int32)
    src_p = jnp.concatenate([src, pad_idx]).reshape(NC, NS, NGR, NG, B)
    dst_p = jnp.concatenate([dst, pad_idx]).reshape(NC, NS, NGR, NG, B)

    x_p = jnp.zeros((R, D), x.dtype).at[:N].set(x)
    zeros1 = jnp.zeros((RD,), jnp.float32)
    zeros2 = jnp.zeros((R, D), jnp.float32)

    degp = _deg_kernel(dst_p, zeros1)
    dg = (degp[0, :R] + degp[1, :R] + 1.0).reshape(R, 1)

    t1 = _tc1(x_p, W1, dg)
    s1 = _spmm_kernel(t1, src_p, dst_p, zeros2)
    t2 = _tc2(s1, dg, b1.reshape(1, D))
    s2 = _spmm_kernel(t2, src_p, dst_p, zeros2)
    wcat = jnp.concatenate([Wmu, Wlv], axis=1)
    bcat = jnp.concatenate([bmu, blv]).reshape(1, D)
    out = _tc3(s2, dg, wcat, bcat)
    return out[:N, : D // 2], out[:N, D // 2 :]


def kernel(x, edge_index, W1, b1, Wmu, bmu, Wlv, blv):
    return _pipeline(x, edge_index, W1, b1, Wmu, bmu, Wlv, blv)
